# baseline (device time: 32331 ns/iter reference)
import jax
import jax.numpy as jnp
from jax import lax
from jax.experimental import pallas as pl
from jax.experimental.pallas import tpu as pltpu

N_CHUNKS = 8


def kernel(x):
    _, m, n_full = x.shape
    n = n_full // 2
    rows = m // N_CHUNKS

    def body(
        x_ref,
        out_ref,
        xchunk,
        send_buf,
        recv_buf,
        out_vmem,
        in_sems,
        send_sems,
        recv_sems,
        out_sems,
    ):
        mx = lax.axis_index("x")
        my = lax.axis_index("y")
        mz = lax.axis_index("z")
        peer_x = 1 - mx

        in_dmas = []
        for c in range(N_CHUNKS):
            dma = pltpu.make_async_copy(
                x_ref.at[0, pl.ds(c * rows, rows), :],
                xchunk.at[c],
                in_sems.at[c],
            )
            dma.start()
            in_dmas.append(dma)

        barrier_sem = pltpu.get_barrier_semaphore()
        pl.semaphore_signal(
            barrier_sem,
            inc=1,
            device_id=(peer_x, my, mz),
            device_id_type=pl.DeviceIdType.MESH,
        )
        pl.semaphore_wait(barrier_sem, 1)

        rdmas = []
        for c in range(N_CHUNKS):
            in_dmas[c].wait()
            send_buf[c] = xchunk[c, :, pl.ds(peer_x * n, n)].astype(jnp.bfloat16)
            rdma = pltpu.make_async_remote_copy(
                src_ref=send_buf.at[c],
                dst_ref=recv_buf.at[c],
                send_sem=send_sems.at[c],
                recv_sem=recv_sems.at[c],
                device_id=(peer_x, my, mz),
                device_id_type=pl.DeviceIdType.MESH,
            )
            rdma.start()
            rdmas.append(rdma)

        out_dmas = []
        for c in range(N_CHUNKS):
            rdmas[c].wait_recv()
            out_vmem[c] = xchunk[c, :, pl.ds(mx * n, n)] + recv_buf[c].astype(
                jnp.float32
            )
            dma = pltpu.make_async_copy(
                out_vmem.at[c],
                out_ref.at[pl.ds(c * rows, rows), :],
                out_sems.at[c],
            )
            dma.start()
            out_dmas.append(dma)

        for c in range(N_CHUNKS):
            rdmas[c].wait_send()
            out_dmas[c].wait()

    return pl.pallas_call(
        body,
        out_shape=jax.ShapeDtypeStruct((m, n), jnp.float32),
        in_specs=[pl.BlockSpec(memory_space=pl.ANY)],
        out_specs=pl.BlockSpec(memory_space=pl.ANY),
        scratch_shapes=[
            pltpu.VMEM((N_CHUNKS, rows, n_full), jnp.float32),
            pltpu.VMEM((N_CHUNKS, rows, n), jnp.bfloat16),
            pltpu.VMEM((N_CHUNKS, rows, n), jnp.bfloat16),
            pltpu.VMEM((N_CHUNKS, rows, n), jnp.float32),
            pltpu.SemaphoreType.DMA((N_CHUNKS,)),
            pltpu.SemaphoreType.DMA((N_CHUNKS,)),
            pltpu.SemaphoreType.DMA((N_CHUNKS,)),
            pltpu.SemaphoreType.DMA((N_CHUNKS,)),
        ],
        compiler_params=pltpu.CompilerParams(collective_id=0),
    )(x)


# device time: 31539 ns/iter; 1.0251x vs baseline; 1.0251x over previous
import jax
import jax.numpy as jnp
from jax import lax
from jax.experimental import pallas as pl
from jax.experimental.pallas import tpu as pltpu

N_CHUNKS = 4


def kernel(x):
    _, m, n_full = x.shape
    n = n_full // 2
    rows = m // N_CHUNKS

    def body(x_ref, out_ref, send_buf, recv_buf, send_sems, recv_sems):
        mx = lax.axis_index("x")
        my = lax.axis_index("y")
        mz = lax.axis_index("z")
        peer_x = 1 - mx

        barrier_sem = pltpu.get_barrier_semaphore()
        pl.semaphore_signal(
            barrier_sem,
            inc=1,
            device_id=(peer_x, my, mz),
            device_id_type=pl.DeviceIdType.MESH,
        )
        pl.semaphore_wait(barrier_sem, 1)

        rdmas = []
        for c in range(N_CHUNKS):
            r0 = c * rows
            send_buf[c] = x_ref[0, pl.ds(r0, rows), pl.ds(peer_x * n, n)].astype(
                jnp.bfloat16
            )
            rdma = pltpu.make_async_remote_copy(
                src_ref=send_buf.at[c],
                dst_ref=recv_buf.at[c],
                send_sem=send_sems.at[c],
                recv_sem=recv_sems.at[c],
                device_id=(peer_x, my, mz),
                device_id_type=pl.DeviceIdType.MESH,
            )
            rdma.start()
            rdmas.append(rdma)

        for c in range(N_CHUNKS):
            rdmas[c].wait_recv()
            r0 = c * rows
            out_ref[pl.ds(r0, rows), :] = x_ref[
                0, pl.ds(r0, rows), pl.ds(mx * n, n)
            ] + recv_buf[c].astype(jnp.float32)

        for c in range(N_CHUNKS):
            rdmas[c].wait_send()

    return pl.pallas_call(
        body,
        out_shape=jax.ShapeDtypeStruct((m, n), jnp.float32),
        in_specs=[pl.BlockSpec(memory_space=pltpu.VMEM)],
        out_specs=pl.BlockSpec(memory_space=pltpu.VMEM),
        scratch_shapes=[
            pltpu.VMEM((N_CHUNKS, rows, n), jnp.bfloat16),
            pltpu.VMEM((N_CHUNKS, rows, n), jnp.bfloat16),
            pltpu.SemaphoreType.DMA((N_CHUNKS,)),
            pltpu.SemaphoreType.DMA((N_CHUNKS,)),
        ],
        compiler_params=pltpu.CompilerParams(collective_id=0),
    )(x)
